# Initial kernel scaffold; baseline (speedup 1.0000x reference)
import jax
import jax.numpy as jnp
from jax.experimental import pallas as pl


def kernel(v_0, t, X, Y, stddevs, approx_flag, alpha_bars):
    raise NotImplementedError("write your pallas kernel here")


# speedup vs baseline: 1.3607x; 1.3607x over previous
import math

import numpy as np
import jax
import jax.numpy as jnp
from jax.experimental import pallas as pl
from jax.experimental.pallas import tpu as pltpu

_NUM_BINS = 8192
_B = _NUM_BINS - 1
_PI = math.pi
_TINY = float(np.finfo(np.float32).tiny)
_FMAX = 3.0e38
_LO = float(np.nextafter(np.float32(-1.0), np.float32(0.0)))
_SPAN = float(np.float32(np.float32(1.0) - np.float32(_LO)))
_SQRT2 = float(np.float32(np.sqrt(2.0)))
_STEP = float(np.float32(_PI / _B))

_TOKT = 64
_CHUNK = 512
_NCH = _NUM_BINS // _CHUNK


def _tf_host(k0, k1, x0, x1):
    msk = 0xFFFFFFFF
    ks = (k0, k1, (k0 ^ k1 ^ 0x1BD11BDA) & msk)
    rot = ((13, 15, 26, 6), (17, 29, 16, 24))
    inj = ((ks[1], ks[2], 1), (ks[2], ks[0], 2), (ks[0], ks[1], 3),
           (ks[1], ks[2], 4), (ks[2], ks[0], 5))
    x0 = (x0 + ks[0]) & msk
    x1 = (x1 + ks[1]) & msk
    for i in range(5):
        for r in rot[i % 2]:
            x0 = (x0 + x1) & msk
            x1 = ((x1 << r) | (x1 >> (32 - r))) & msk
            x1 ^= x0
        a, b, c = inj[i]
        x0 = (x0 + a) & msk
        x1 = (x1 + b + c) & msk
    return x0, x1


def _split_host(key, num):
    return [_tf_host(key[0], key[1], 0, i) for i in range(num)]


_KEY_ROOT = (0, 1)
_KU, _KT = _split_host(_KEY_ROOT, 2)
_K1, _K2, _K3 = _split_host(_KT, 3)


def _i32c(v):
    return jnp.int32(int(np.uint32(v & 0xFFFFFFFF).astype(np.int32)))


def _threefry_bits(key, x1):
    k0, k1 = key
    ks2 = (k0 ^ k1 ^ 0x1BD11BDA) & 0xFFFFFFFF
    rot = ((13, 15, 26, 6), (17, 29, 16, 24))
    inj = ((k1, ks2, 1), (ks2, k0, 2), (k0, k1, 3), (k1, ks2, 4), (ks2, k0, 5))
    x0 = jnp.full(x1.shape, _i32c(k0), jnp.int32)
    x1 = x1 + _i32c(k1)
    for i in range(5):
        for r in rot[i % 2]:
            x0 = x0 + x1
            x1 = jax.lax.shift_left(x1, jnp.int32(r)) | \
                jax.lax.shift_right_logical(x1, jnp.int32(32 - r))
            x1 = x1 ^ x0
        a, b, c = inj[i]
        x0 = x0 + _i32c(a)
        x1 = x1 + _i32c(b + c)
    return x0 ^ x1


def _bits_to_unit(bits):
    fb = jax.lax.shift_right_logical(bits, jnp.int32(9)) | jnp.int32(0x3F800000)
    return jax.lax.bitcast_convert_type(fb, jnp.float32) - jnp.float32(1.0)


def _horner(w, coeffs):
    p = jnp.float32(coeffs[0])
    for c in coeffs[1:]:
        p = p * w + jnp.float32(c)
    return p


_ERFINV_SMALL = (2.81022636e-08, 3.43273939e-07, -3.5233877e-06,
                 -4.39150654e-06, 0.00021858087, -0.00125372503,
                 -0.00417768164, 0.246640727, 1.50140941)
_ERFINV_BIG = (-0.000200214257, 0.000100950558, 0.00134934322,
               -0.00367342844, 0.00573950773, -0.0076224613,
               0.00943887047, 1.00167406, 2.83297682)


def _erfinv(x):
    w = -jnp.log((jnp.float32(1.0) - x) * (jnp.float32(1.0) + x))
    small = w < jnp.float32(5.0)
    ws = w - jnp.float32(2.5)
    wb = jnp.sqrt(jnp.where(small, jnp.float32(5.0), w)) - jnp.float32(3.0)
    p = jnp.where(small, _horner(ws, _ERFINV_SMALL), _horner(wb, _ERFINV_BIG))
    return p * x


_ACOS_C = (-0.0012624911, 0.0066700901, -0.0170881256, 0.0308918810,
           -0.0501743046, 0.0889789874, -0.2145988016, 1.5707963050)


def _acos(x):
    ax = jnp.abs(x)
    p = _horner(ax, _ACOS_C)
    r = jnp.sqrt(jnp.maximum(jnp.float32(1.0) - ax, jnp.float32(0.0))) * p
    return jnp.where(x >= 0, r, jnp.float32(_PI) - r)


def _std_normal(key, m):
    f = _bits_to_unit(_threefry_bits(key, m))
    u = jnp.maximum(f * jnp.float32(_SPAN) + jnp.float32(_LO), jnp.float32(_LO))
    return jnp.float32(_SQRT2) * _erfinv(u)


def _bf(x):
    return x.astype(jnp.bfloat16).astype(jnp.float32)


def _rodrigues(x, y, z):
    n = jnp.sqrt(x * x + y * y + z * z)
    s = jnp.sin(n)
    ct = jnp.cos(n)
    b = (s + jnp.float32(1e-8)) / (n + jnp.float32(1e-8))
    c = (jnp.float32(1.0) - ct + jnp.float32(1e-8)) / (n * n + jnp.float32(2e-8))
    xb = _bf(x)
    yb = _bf(y)
    zb = _bf(z)
    s2xy = xb * yb
    s2xz = xb * zb
    s2yz = yb * zb
    r00 = 1.0 + c * (-(zb * zb) - yb * yb)
    r01 = b * z + c * s2xy
    r02 = -(b * y) + c * s2xz
    r10 = -(b * z) + c * s2xy
    r11 = 1.0 + c * (-(zb * zb) - xb * xb)
    r12 = b * x + c * s2yz
    r20 = b * y + c * s2xz
    r21 = -(b * x) + c * s2yz
    r22 = 1.0 + c * (-(yb * yb) - xb * xb)
    return (r00, r01, r02, r10, r11, r12, r20, r21, r22)


def _cat_kernel(t_ref, y_ref, bidx_ref, bw_ref, bb_ref):
    p = pl.program_id(0)
    k = pl.program_id(1)
    l = jax.lax.broadcasted_iota(jnp.int32, (_TOKT, _CHUNK), 0)
    j = jax.lax.broadcasted_iota(jnp.int32, (_TOKT, _CHUNK), 1)
    gbin = k * _CHUNK + j
    m = (p * _TOKT + l) * _B + gbin
    bits = _threefry_bits(_K1, m)
    u = jnp.maximum(_bits_to_unit(bits), jnp.float32(_TINY))
    nl = jnp.float32(0.0) - jnp.log(u)
    recip = jnp.float32(1.0) / (y_ref[0] + jnp.float32(1e-20))
    w = nl * recip
    w = jnp.where(gbin >= _B, jnp.float32(_FMAX), w)

    @pl.when(k == 0)
    def _init():
        bw_ref[...] = jnp.full((_TOKT, _CHUNK), _FMAX, jnp.float32)
        bb_ref[...] = jnp.zeros((_TOKT, _CHUNK), jnp.int32)

    upd = w < bw_ref[...]
    bb_ref[...] = jnp.where(upd, gbin, bb_ref[...])
    bw_ref[...] = jnp.where(upd, w, bw_ref[...])

    @pl.when(k == _NCH - 1)
    def _finish():
        wv = bw_ref[...]
        mv = jnp.min(wv, axis=1, keepdims=True)
        cand = jnp.where(wv == mv, bb_ref[...], jnp.int32(1 << 30))
        bidx_ref[0] = jnp.min(cand, axis=1, keepdims=True)


def _epi_kernel(bidx_ref, flag_ref, std_ref, ab_ref, v0x_ref, v0y_ref, v0z_ref,
                vx_ref, vy_ref, vz_ref, ex_ref, ey_ref, ez_ref):
    shape = (64, 128)
    i = jax.lax.broadcasted_iota(jnp.int32, shape, 0)
    j = jax.lax.broadcasted_iota(jnp.int32, shape, 1)
    r = i * 128 + j

    u2 = _bits_to_unit(_threefry_bits(_K2, r))
    bf = bidx_ref[...].astype(jnp.float32)
    smp_h = bf * jnp.float32(_STEP) + u2 * jnp.float32(_STEP)

    n3 = _std_normal(_K3, r)
    std = std_ref[...]
    smp_g = jax.lax.rem(jnp.abs(std * jnp.float32(2.0) + n3 * std),
                        jnp.float32(_PI))
    theta = jnp.where(flag_ref[...] > jnp.float32(0.5), smp_g, smp_h)

    m0 = r * 3
    g0 = _std_normal(_KU, m0)
    g1 = _std_normal(_KU, m0 + 1)
    g2 = _std_normal(_KU, m0 + 2)
    nrm = jnp.sqrt(g0 * g0 + g1 * g1 + g2 * g2) + jnp.float32(1e-12)
    ex = (g0 / nrm) * theta
    ey = (g1 / nrm) * theta
    ez = (g2 / nrm) * theta
    ex_ref[...] = ex
    ey_ref[...] = ey
    ez_ref[...] = ez

    E = [_bf(v) for v in _rodrigues(ex, ey, ez)]
    c0 = jnp.sqrt(ab_ref[...])
    A = [_bf(v) for v in _rodrigues(
        c0 * v0x_ref[...], c0 * v0y_ref[...], c0 * v0z_ref[...])]
    e00, e01, e02, e10, e11, e12, e20, e21, e22 = E
    a00, a01, a02, a10, a11, a12, a20, a21, a22 = A
    r00 = e00 * a00 + e01 * a10 + e02 * a20
    r01 = e00 * a01 + e01 * a11 + e02 * a21
    r02 = e00 * a02 + e01 * a12 + e02 * a22
    r10 = e10 * a00 + e11 * a10 + e12 * a20
    r11 = e10 * a01 + e11 * a11 + e12 * a21
    r12 = e10 * a02 + e11 * a12 + e12 * a22
    r20 = e20 * a00 + e21 * a10 + e22 * a20
    r21 = e20 * a01 + e21 * a11 + e22 * a21
    r22 = e20 * a02 + e21 * a12 + e22 * a22

    tr = r00 + r11 + r22
    ct = jnp.clip((tr - jnp.float32(1.0)) / jnp.float32(2.0),
                  jnp.float32(-0.999), jnp.float32(1.0))
    st = jnp.sqrt(jnp.float32(1.0) - ct * ct)
    th = _acos(ct)
    coef = (th + jnp.float32(1e-8)) / (jnp.float32(2.0) * st + jnp.float32(2e-8))
    vx_ref[...] = coef * (r12 - r21)
    vy_ref[...] = coef * (r20 - r02)
    vz_ref[...] = coef * (r01 - r10)


def kernel(v_0, t, X, Y, stddevs, approx_flag, alpha_bars):
    N, L = v_0.shape[:2]
    nt = N * L
    t = t.astype(jnp.int32)
    tiles = nt // _TOKT

    y_rows = Y.reshape(Y.shape[0] * _NCH, 1, _CHUNK)

    grid_spec = pltpu.PrefetchScalarGridSpec(
        num_scalar_prefetch=1,
        grid=(tiles, _NCH),
        in_specs=[
            pl.BlockSpec(
                (1, 1, _CHUNK),
                lambda p, k, t_ref: (t_ref[(p * _TOKT) // L] * _NCH + k, 0, 0)),
        ],
        out_specs=pl.BlockSpec((1, _TOKT, 1), lambda p, k, t_ref: (p, 0, 0)),
        scratch_shapes=[
            pltpu.VMEM((_TOKT, _CHUNK), jnp.float32),
            pltpu.VMEM((_TOKT, _CHUNK), jnp.int32),
        ],
    )
    bidx = pl.pallas_call(
        _cat_kernel,
        grid_spec=grid_spec,
        out_shape=jax.ShapeDtypeStruct((tiles, _TOKT, 1), jnp.int32),
        compiler_params=pltpu.CompilerParams(
            dimension_semantics=("arbitrary", "arbitrary")),
    )(t, y_rows)

    plane = (nt // 128, 128)
    bidx_p = bidx.reshape(plane)
    flag_p = jnp.broadcast_to(
        approx_flag[t].astype(jnp.float32)[:, None], (N, L)).reshape(plane)
    std_p = jnp.broadcast_to(stddevs[t][:, None], (N, L)).reshape(plane)
    ab_p = jnp.broadcast_to(alpha_bars[t][:, None], (N, L)).reshape(plane)
    v0x = v_0[..., 0].reshape(plane)
    v0y = v_0[..., 1].reshape(plane)
    v0z = v_0[..., 2].reshape(plane)

    outs = pl.pallas_call(
        _epi_kernel,
        out_shape=[jax.ShapeDtypeStruct(plane, jnp.float32)] * 6,
    )(bidx_p, flag_p, std_p, ab_p, v0x, v0y, v0z)
    vx, vy, vz, ex, ey, ez = outs
    v_noisy = jnp.stack(
        [vx.reshape(nt), vy.reshape(nt), vz.reshape(nt)], axis=-1
    ).reshape(N, L, 3)
    e_scaled = jnp.stack(
        [ex.reshape(nt), ey.reshape(nt), ez.reshape(nt)], axis=-1
    ).reshape(N, L, 3)
    return v_noisy, e_scaled


# speedup vs baseline: 1.6313x; 1.1989x over previous
import math

import numpy as np
import jax
import jax.numpy as jnp
from jax.experimental import pallas as pl
from jax.experimental.pallas import tpu as pltpu

_NUM_BINS = 8192
_B = _NUM_BINS - 1
_PI = math.pi
_TINY = float(np.finfo(np.float32).tiny)
_FMAX = 3.0e38
_LO = float(np.nextafter(np.float32(-1.0), np.float32(0.0)))
_SPAN = float(np.float32(np.float32(1.0) - np.float32(_LO)))
_SQRT2 = float(np.float32(np.sqrt(2.0)))
_STEP = float(np.float32(_PI / _B))

_TOKT = 64
_CHUNK = 1024
_NCH = _NUM_BINS // _CHUNK


def _tf_host(k0, k1, x0, x1):
    msk = 0xFFFFFFFF
    ks = (k0, k1, (k0 ^ k1 ^ 0x1BD11BDA) & msk)
    rot = ((13, 15, 26, 6), (17, 29, 16, 24))
    inj = ((ks[1], ks[2], 1), (ks[2], ks[0], 2), (ks[0], ks[1], 3),
           (ks[1], ks[2], 4), (ks[2], ks[0], 5))
    x0 = (x0 + ks[0]) & msk
    x1 = (x1 + ks[1]) & msk
    for i in range(5):
        for r in rot[i % 2]:
            x0 = (x0 + x1) & msk
            x1 = ((x1 << r) | (x1 >> (32 - r))) & msk
            x1 ^= x0
        a, b, c = inj[i]
        x0 = (x0 + a) & msk
        x1 = (x1 + b + c) & msk
    return x0, x1


def _split_host(key, num):
    return [_tf_host(key[0], key[1], 0, i) for i in range(num)]


_KEY_ROOT = (0, 1)
_KU, _KT = _split_host(_KEY_ROOT, 2)
_K1, _K2, _K3 = _split_host(_KT, 3)


def _i32c(v):
    return jnp.int32(int(np.uint32(v & 0xFFFFFFFF).astype(np.int32)))


def _threefry_bits(key, x1):
    k0, k1 = key
    ks2 = (k0 ^ k1 ^ 0x1BD11BDA) & 0xFFFFFFFF
    rot = ((13, 15, 26, 6), (17, 29, 16, 24))
    inj = ((k1, ks2, 1), (ks2, k0, 2), (k0, k1, 3), (k1, ks2, 4), (ks2, k0, 5))
    x0 = jnp.full(x1.shape, _i32c(k0), jnp.int32)
    x1 = x1 + _i32c(k1)
    for i in range(5):
        for r in rot[i % 2]:
            x0 = x0 + x1
            x1 = jax.lax.shift_left(x1, jnp.int32(r)) | \
                jax.lax.shift_right_logical(x1, jnp.int32(32 - r))
            x1 = x1 ^ x0
        a, b, c = inj[i]
        x0 = x0 + _i32c(a)
        x1 = x1 + _i32c(b + c)
    return x0 ^ x1


def _bits_to_unit(bits):
    fb = jax.lax.shift_right_logical(bits, jnp.int32(9)) | jnp.int32(0x3F800000)
    return jax.lax.bitcast_convert_type(fb, jnp.float32) - jnp.float32(1.0)


def _horner(w, coeffs):
    p = jnp.float32(coeffs[0])
    for c in coeffs[1:]:
        p = p * w + jnp.float32(c)
    return p


_ERFINV_SMALL = (2.81022636e-08, 3.43273939e-07, -3.5233877e-06,
                 -4.39150654e-06, 0.00021858087, -0.00125372503,
                 -0.00417768164, 0.246640727, 1.50140941)
_ERFINV_BIG = (-0.000200214257, 0.000100950558, 0.00134934322,
               -0.00367342844, 0.00573950773, -0.0076224613,
               0.00943887047, 1.00167406, 2.83297682)


def _erfinv(x):
    w = -jnp.log((jnp.float32(1.0) - x) * (jnp.float32(1.0) + x))
    small = w < jnp.float32(5.0)
    ws = w - jnp.float32(2.5)
    wb = jnp.sqrt(jnp.where(small, jnp.float32(5.0), w)) - jnp.float32(3.0)
    p = jnp.where(small, _horner(ws, _ERFINV_SMALL), _horner(wb, _ERFINV_BIG))
    return p * x


_ACOS_C = (-0.0012624911, 0.0066700901, -0.0170881256, 0.0308918810,
           -0.0501743046, 0.0889789874, -0.2145988016, 1.5707963050)


def _acos(x):
    ax = jnp.abs(x)
    p = _horner(ax, _ACOS_C)
    r = jnp.sqrt(jnp.maximum(jnp.float32(1.0) - ax, jnp.float32(0.0))) * p
    return jnp.where(x >= 0, r, jnp.float32(_PI) - r)


def _std_normal(key, m):
    f = _bits_to_unit(_threefry_bits(key, m))
    u = jnp.maximum(f * jnp.float32(_SPAN) + jnp.float32(_LO), jnp.float32(_LO))
    return jnp.float32(_SQRT2) * _erfinv(u)


def _bf(x):
    return x.astype(jnp.bfloat16).astype(jnp.float32)


def _rodrigues(x, y, z):
    n = jnp.sqrt(x * x + y * y + z * z)
    s = jnp.sin(n)
    ct = jnp.cos(n)
    b = (s + jnp.float32(1e-8)) / (n + jnp.float32(1e-8))
    c = (jnp.float32(1.0) - ct + jnp.float32(1e-8)) / (n * n + jnp.float32(2e-8))
    xb = _bf(x)
    yb = _bf(y)
    zb = _bf(z)
    s2xy = xb * yb
    s2xz = xb * zb
    s2yz = yb * zb
    r00 = 1.0 + c * (-(zb * zb) - yb * yb)
    r01 = b * z + c * s2xy
    r02 = -(b * y) + c * s2xz
    r10 = -(b * z) + c * s2xy
    r11 = 1.0 + c * (-(zb * zb) - xb * xb)
    r12 = b * x + c * s2yz
    r20 = b * y + c * s2xz
    r21 = -(b * x) + c * s2yz
    r22 = 1.0 + c * (-(yb * yb) - xb * xb)
    return (r00, r01, r02, r10, r11, r12, r20, r21, r22)


_SLICE = 8


def _cat_kernel(t_ref, y_ref, bidx_ref, bw_ref, bb_ref):
    p = pl.program_id(0)
    k = pl.program_id(1)
    recip = jnp.float32(1.0) / (y_ref[0] + jnp.float32(1e-20))

    @pl.when(k == 0)
    def _init():
        bw_ref[...] = jnp.full((_TOKT, _CHUNK), _FMAX, jnp.float32)
        bb_ref[...] = jnp.zeros((_TOKT, _CHUNK), jnp.int32)

    for ss in range(_TOKT // _SLICE):
        l = jax.lax.broadcasted_iota(jnp.int32, (_SLICE, _CHUNK), 0) + ss * _SLICE
        j = jax.lax.broadcasted_iota(jnp.int32, (_SLICE, _CHUNK), 1)
        gbin = k * _CHUNK + j
        m = (p * _TOKT + l) * _B + gbin
        bits = _threefry_bits(_K1, m)
        u = jnp.maximum(_bits_to_unit(bits), jnp.float32(_TINY))
        w = (jnp.float32(0.0) - jnp.log(u)) * recip
        w = jnp.where(gbin >= _B, jnp.float32(_FMAX), w)
        sl = slice(ss * _SLICE, (ss + 1) * _SLICE)
        bwv = bw_ref[sl, :]
        bbv = bb_ref[sl, :]
        upd = w < bwv
        bb_ref[sl, :] = jnp.where(upd, gbin, bbv)
        bw_ref[sl, :] = jnp.where(upd, w, bwv)

    @pl.when(k == _NCH - 1)
    def _finish():
        wv = bw_ref[...]
        mv = jnp.min(wv, axis=1, keepdims=True)
        cand = jnp.where(wv == mv, bb_ref[...], jnp.int32(1 << 30))
        bidx_ref[0] = jnp.min(cand, axis=1, keepdims=True)


def _epi_kernel(bidx_ref, flag_ref, std_ref, ab_ref, v0x_ref, v0y_ref, v0z_ref,
                vx_ref, vy_ref, vz_ref, ex_ref, ey_ref, ez_ref):
    shape = (64, 128)
    i = jax.lax.broadcasted_iota(jnp.int32, shape, 0)
    j = jax.lax.broadcasted_iota(jnp.int32, shape, 1)
    r = i * 128 + j

    u2 = _bits_to_unit(_threefry_bits(_K2, r))
    bf = bidx_ref[...].astype(jnp.float32)
    smp_h = bf * jnp.float32(_STEP) + u2 * jnp.float32(_STEP)

    n3 = _std_normal(_K3, r)
    std = std_ref[...]
    smp_g = jax.lax.rem(jnp.abs(std * jnp.float32(2.0) + n3 * std),
                        jnp.float32(_PI))
    theta = jnp.where(flag_ref[...] > jnp.float32(0.5), smp_g, smp_h)

    m0 = r * 3
    g0 = _std_normal(_KU, m0)
    g1 = _std_normal(_KU, m0 + 1)
    g2 = _std_normal(_KU, m0 + 2)
    nrm = jnp.sqrt(g0 * g0 + g1 * g1 + g2 * g2) + jnp.float32(1e-12)
    ex = (g0 / nrm) * theta
    ey = (g1 / nrm) * theta
    ez = (g2 / nrm) * theta
    ex_ref[...] = ex
    ey_ref[...] = ey
    ez_ref[...] = ez

    E = [_bf(v) for v in _rodrigues(ex, ey, ez)]
    c0 = jnp.sqrt(ab_ref[...])
    A = [_bf(v) for v in _rodrigues(
        c0 * v0x_ref[...], c0 * v0y_ref[...], c0 * v0z_ref[...])]
    e00, e01, e02, e10, e11, e12, e20, e21, e22 = E
    a00, a01, a02, a10, a11, a12, a20, a21, a22 = A
    r00 = e00 * a00 + e01 * a10 + e02 * a20
    r01 = e00 * a01 + e01 * a11 + e02 * a21
    r02 = e00 * a02 + e01 * a12 + e02 * a22
    r10 = e10 * a00 + e11 * a10 + e12 * a20
    r11 = e10 * a01 + e11 * a11 + e12 * a21
    r12 = e10 * a02 + e11 * a12 + e12 * a22
    r20 = e20 * a00 + e21 * a10 + e22 * a20
    r21 = e20 * a01 + e21 * a11 + e22 * a21
    r22 = e20 * a02 + e21 * a12 + e22 * a22

    tr = r00 + r11 + r22
    ct = jnp.clip((tr - jnp.float32(1.0)) / jnp.float32(2.0),
                  jnp.float32(-0.999), jnp.float32(1.0))
    st = jnp.sqrt(jnp.float32(1.0) - ct * ct)
    th = _acos(ct)
    coef = (th + jnp.float32(1e-8)) / (jnp.float32(2.0) * st + jnp.float32(2e-8))
    vx_ref[...] = coef * (r12 - r21)
    vy_ref[...] = coef * (r20 - r02)
    vz_ref[...] = coef * (r01 - r10)


def kernel(v_0, t, X, Y, stddevs, approx_flag, alpha_bars):
    N, L = v_0.shape[:2]
    nt = N * L
    t = t.astype(jnp.int32)
    tiles = nt // _TOKT

    y_rows = Y.reshape(Y.shape[0] * _NCH, 1, _CHUNK)

    grid_spec = pltpu.PrefetchScalarGridSpec(
        num_scalar_prefetch=1,
        grid=(tiles, _NCH),
        in_specs=[
            pl.BlockSpec(
                (1, 1, _CHUNK),
                lambda p, k, t_ref: (t_ref[(p * _TOKT) // L] * _NCH + k, 0, 0)),
        ],
        out_specs=pl.BlockSpec((1, _TOKT, 1), lambda p, k, t_ref: (p, 0, 0)),
        scratch_shapes=[
            pltpu.VMEM((_TOKT, _CHUNK), jnp.float32),
            pltpu.VMEM((_TOKT, _CHUNK), jnp.int32),
        ],
    )
    bidx = pl.pallas_call(
        _cat_kernel,
        grid_spec=grid_spec,
        out_shape=jax.ShapeDtypeStruct((tiles, _TOKT, 1), jnp.int32),
        compiler_params=pltpu.CompilerParams(
            dimension_semantics=("arbitrary", "arbitrary")),
    )(t, y_rows)

    plane = (nt // 128, 128)
    bidx_p = bidx.reshape(plane)
    flag_p = jnp.broadcast_to(
        approx_flag[t].astype(jnp.float32)[:, None], (N, L)).reshape(plane)
    std_p = jnp.broadcast_to(stddevs[t][:, None], (N, L)).reshape(plane)
    ab_p = jnp.broadcast_to(alpha_bars[t][:, None], (N, L)).reshape(plane)
    v0x = v_0[..., 0].reshape(plane)
    v0y = v_0[..., 1].reshape(plane)
    v0z = v_0[..., 2].reshape(plane)

    outs = pl.pallas_call(
        _epi_kernel,
        out_shape=[jax.ShapeDtypeStruct(plane, jnp.float32)] * 6,
    )(bidx_p, flag_p, std_p, ab_p, v0x, v0y, v0z)
    vx, vy, vz, ex, ey, ez = outs
    v_noisy = jnp.stack(
        [vx.reshape(nt), vy.reshape(nt), vz.reshape(nt)], axis=-1
    ).reshape(N, L, 3)
    e_scaled = jnp.stack(
        [ex.reshape(nt), ey.reshape(nt), ez.reshape(nt)], axis=-1
    ).reshape(N, L, 3)
    return v_noisy, e_scaled


# speedup vs baseline: 1.6958x; 1.0396x over previous
import math

import numpy as np
import jax
import jax.numpy as jnp
from jax.experimental import pallas as pl
from jax.experimental.pallas import tpu as pltpu

_NUM_BINS = 8192
_B = _NUM_BINS - 1
_PI = math.pi
_TINY = float(np.finfo(np.float32).tiny)
_FMAX = 3.0e38
_LO = float(np.nextafter(np.float32(-1.0), np.float32(0.0)))
_SPAN = float(np.float32(np.float32(1.0) - np.float32(_LO)))
_SQRT2 = float(np.float32(np.sqrt(2.0)))
_STEP = float(np.float32(_PI / _B))

_TOKT = 64
_CHUNK = 4096
_NCH = _NUM_BINS // _CHUNK


def _tf_host(k0, k1, x0, x1):
    msk = 0xFFFFFFFF
    ks = (k0, k1, (k0 ^ k1 ^ 0x1BD11BDA) & msk)
    rot = ((13, 15, 26, 6), (17, 29, 16, 24))
    inj = ((ks[1], ks[2], 1), (ks[2], ks[0], 2), (ks[0], ks[1], 3),
           (ks[1], ks[2], 4), (ks[2], ks[0], 5))
    x0 = (x0 + ks[0]) & msk
    x1 = (x1 + ks[1]) & msk
    for i in range(5):
        for r in rot[i % 2]:
            x0 = (x0 + x1) & msk
            x1 = ((x1 << r) | (x1 >> (32 - r))) & msk
            x1 ^= x0
        a, b, c = inj[i]
        x0 = (x0 + a) & msk
        x1 = (x1 + b + c) & msk
    return x0, x1


def _split_host(key, num):
    return [_tf_host(key[0], key[1], 0, i) for i in range(num)]


_KEY_ROOT = (0, 1)
_KU, _KT = _split_host(_KEY_ROOT, 2)
_K1, _K2, _K3 = _split_host(_KT, 3)


def _i32c(v):
    return jnp.int32(int(np.uint32(v & 0xFFFFFFFF).astype(np.int32)))


def _threefry_bits(key, x1):
    k0, k1 = key
    ks2 = (k0 ^ k1 ^ 0x1BD11BDA) & 0xFFFFFFFF
    rot = ((13, 15, 26, 6), (17, 29, 16, 24))
    inj = ((k1, ks2, 1), (ks2, k0, 2), (k0, k1, 3), (k1, ks2, 4), (ks2, k0, 5))
    x0 = jnp.full(x1.shape, _i32c(k0), jnp.int32)
    x1 = x1 + _i32c(k1)
    for i in range(5):
        for r in rot[i % 2]:
            x0 = x0 + x1
            x1 = jax.lax.shift_left(x1, jnp.int32(r)) | \
                jax.lax.shift_right_logical(x1, jnp.int32(32 - r))
            x1 = x1 ^ x0
        a, b, c = inj[i]
        x0 = x0 + _i32c(a)
        x1 = x1 + _i32c(b + c)
    return x0 ^ x1


def _bits_to_unit(bits):
    fb = jax.lax.shift_right_logical(bits, jnp.int32(9)) | jnp.int32(0x3F800000)
    return jax.lax.bitcast_convert_type(fb, jnp.float32) - jnp.float32(1.0)


def _horner(w, coeffs):
    p = jnp.float32(coeffs[0])
    for c in coeffs[1:]:
        p = p * w + jnp.float32(c)
    return p


_ERFINV_SMALL = (2.81022636e-08, 3.43273939e-07, -3.5233877e-06,
                 -4.39150654e-06, 0.00021858087, -0.00125372503,
                 -0.00417768164, 0.246640727, 1.50140941)
_ERFINV_BIG = (-0.000200214257, 0.000100950558, 0.00134934322,
               -0.00367342844, 0.00573950773, -0.0076224613,
               0.00943887047, 1.00167406, 2.83297682)


def _erfinv(x):
    w = -jnp.log((jnp.float32(1.0) - x) * (jnp.float32(1.0) + x))
    small = w < jnp.float32(5.0)
    ws = w - jnp.float32(2.5)
    wb = jnp.sqrt(jnp.where(small, jnp.float32(5.0), w)) - jnp.float32(3.0)
    p = jnp.where(small, _horner(ws, _ERFINV_SMALL), _horner(wb, _ERFINV_BIG))
    return p * x


_ACOS_C = (-0.0012624911, 0.0066700901, -0.0170881256, 0.0308918810,
           -0.0501743046, 0.0889789874, -0.2145988016, 1.5707963050)


def _acos(x):
    ax = jnp.abs(x)
    p = _horner(ax, _ACOS_C)
    r = jnp.sqrt(jnp.maximum(jnp.float32(1.0) - ax, jnp.float32(0.0))) * p
    return jnp.where(x >= 0, r, jnp.float32(_PI) - r)


def _std_normal(key, m):
    f = _bits_to_unit(_threefry_bits(key, m))
    u = jnp.maximum(f * jnp.float32(_SPAN) + jnp.float32(_LO), jnp.float32(_LO))
    return jnp.float32(_SQRT2) * _erfinv(u)


def _bf(x):
    return x.astype(jnp.bfloat16).astype(jnp.float32)


def _rodrigues(x, y, z):
    n = jnp.sqrt(x * x + y * y + z * z)
    s = jnp.sin(n)
    ct = jnp.cos(n)
    b = (s + jnp.float32(1e-8)) / (n + jnp.float32(1e-8))
    c = (jnp.float32(1.0) - ct + jnp.float32(1e-8)) / (n * n + jnp.float32(2e-8))
    xb = _bf(x)
    yb = _bf(y)
    zb = _bf(z)
    s2xy = xb * yb
    s2xz = xb * zb
    s2yz = yb * zb
    r00 = 1.0 + c * (-(zb * zb) - yb * yb)
    r01 = b * z + c * s2xy
    r02 = -(b * y) + c * s2xz
    r10 = -(b * z) + c * s2xy
    r11 = 1.0 + c * (-(zb * zb) - xb * xb)
    r12 = b * x + c * s2yz
    r20 = b * y + c * s2xz
    r21 = -(b * x) + c * s2yz
    r22 = 1.0 + c * (-(yb * yb) - xb * xb)
    return (r00, r01, r02, r10, r11, r12, r20, r21, r22)


_SLICE = 8


def _cat_kernel(t_ref, y_ref, bidx_ref, bw_ref, bb_ref):
    p = pl.program_id(0)
    k = pl.program_id(1)
    jrow = jax.lax.broadcasted_iota(jnp.int32, (1, _CHUNK), 1)
    nrecip = jnp.float32(-1.0) / (y_ref[0] + jnp.float32(1e-20))
    nrecip = jnp.where(k * _CHUNK + jrow >= _B, jnp.float32(-_FMAX), nrecip)

    @pl.when(k == 0)
    def _init():
        bw_ref[...] = jnp.full((_TOKT, _CHUNK), _FMAX, jnp.float32)
        bb_ref[...] = jnp.zeros((_TOKT, _CHUNK), jnp.int32)

    for ss in range(_TOKT // _SLICE):
        l = jax.lax.broadcasted_iota(jnp.int32, (_SLICE, _CHUNK), 0) + ss * _SLICE
        j = jax.lax.broadcasted_iota(jnp.int32, (_SLICE, _CHUNK), 1)
        gbin = k * _CHUNK + j
        m = (p * _TOKT + l) * _B + gbin
        bits = _threefry_bits(_K1, m)
        u = jnp.maximum(_bits_to_unit(bits), jnp.float32(_TINY))
        w = jnp.log(u) * nrecip
        sl = slice(ss * _SLICE, (ss + 1) * _SLICE)
        bwv = bw_ref[sl, :]
        bbv = bb_ref[sl, :]
        upd = w < bwv
        bb_ref[sl, :] = jnp.where(upd, gbin, bbv)
        bw_ref[sl, :] = jnp.where(upd, w, bwv)

    @pl.when(k == _NCH - 1)
    def _finish():
        wv = bw_ref[...]
        mv = jnp.min(wv, axis=1, keepdims=True)
        cand = jnp.where(wv == mv, bb_ref[...], jnp.int32(1 << 30))
        bidx_ref[0] = jnp.min(cand, axis=1, keepdims=True)


def _epi_kernel(bidx_ref, flag_ref, std_ref, ab_ref, v0x_ref, v0y_ref, v0z_ref,
                vx_ref, vy_ref, vz_ref, ex_ref, ey_ref, ez_ref):
    shape = (64, 128)
    i = jax.lax.broadcasted_iota(jnp.int32, shape, 0)
    j = jax.lax.broadcasted_iota(jnp.int32, shape, 1)
    r = i * 128 + j

    u2 = _bits_to_unit(_threefry_bits(_K2, r))
    bf = bidx_ref[...].astype(jnp.float32)
    smp_h = bf * jnp.float32(_STEP) + u2 * jnp.float32(_STEP)

    n3 = _std_normal(_K3, r)
    std = std_ref[...]
    smp_g = jax.lax.rem(jnp.abs(std * jnp.float32(2.0) + n3 * std),
                        jnp.float32(_PI))
    theta = jnp.where(flag_ref[...] > jnp.float32(0.5), smp_g, smp_h)

    m0 = r * 3
    g0 = _std_normal(_KU, m0)
    g1 = _std_normal(_KU, m0 + 1)
    g2 = _std_normal(_KU, m0 + 2)
    nrm = jnp.sqrt(g0 * g0 + g1 * g1 + g2 * g2) + jnp.float32(1e-12)
    ex = (g0 / nrm) * theta
    ey = (g1 / nrm) * theta
    ez = (g2 / nrm) * theta
    ex_ref[...] = ex
    ey_ref[...] = ey
    ez_ref[...] = ez

    E = [_bf(v) for v in _rodrigues(ex, ey, ez)]
    c0 = jnp.sqrt(ab_ref[...])
    A = [_bf(v) for v in _rodrigues(
        c0 * v0x_ref[...], c0 * v0y_ref[...], c0 * v0z_ref[...])]
    e00, e01, e02, e10, e11, e12, e20, e21, e22 = E
    a00, a01, a02, a10, a11, a12, a20, a21, a22 = A
    r00 = e00 * a00 + e01 * a10 + e02 * a20
    r01 = e00 * a01 + e01 * a11 + e02 * a21
    r02 = e00 * a02 + e01 * a12 + e02 * a22
    r10 = e10 * a00 + e11 * a10 + e12 * a20
    r11 = e10 * a01 + e11 * a11 + e12 * a21
    r12 = e10 * a02 + e11 * a12 + e12 * a22
    r20 = e20 * a00 + e21 * a10 + e22 * a20
    r21 = e20 * a01 + e21 * a11 + e22 * a21
    r22 = e20 * a02 + e21 * a12 + e22 * a22

    tr = r00 + r11 + r22
    ct = jnp.clip((tr - jnp.float32(1.0)) / jnp.float32(2.0),
                  jnp.float32(-0.999), jnp.float32(1.0))
    st = jnp.sqrt(jnp.float32(1.0) - ct * ct)
    th = _acos(ct)
    coef = (th + jnp.float32(1e-8)) / (jnp.float32(2.0) * st + jnp.float32(2e-8))
    vx_ref[...] = coef * (r12 - r21)
    vy_ref[...] = coef * (r20 - r02)
    vz_ref[...] = coef * (r01 - r10)


def kernel(v_0, t, X, Y, stddevs, approx_flag, alpha_bars):
    N, L = v_0.shape[:2]
    nt = N * L
    t = t.astype(jnp.int32)
    tiles = nt // _TOKT

    y_rows = Y.reshape(Y.shape[0] * _NCH, 1, _CHUNK)

    grid_spec = pltpu.PrefetchScalarGridSpec(
        num_scalar_prefetch=1,
        grid=(tiles, _NCH),
        in_specs=[
            pl.BlockSpec(
                (1, 1, _CHUNK),
                lambda p, k, t_ref: (t_ref[(p * _TOKT) // L] * _NCH + k, 0, 0)),
        ],
        out_specs=pl.BlockSpec((1, _TOKT, 1), lambda p, k, t_ref: (p, 0, 0)),
        scratch_shapes=[
            pltpu.VMEM((_TOKT, _CHUNK), jnp.float32),
            pltpu.VMEM((_TOKT, _CHUNK), jnp.int32),
        ],
    )
    bidx = pl.pallas_call(
        _cat_kernel,
        grid_spec=grid_spec,
        out_shape=jax.ShapeDtypeStruct((tiles, _TOKT, 1), jnp.int32),
        compiler_params=pltpu.CompilerParams(
            dimension_semantics=("arbitrary", "arbitrary")),
    )(t, y_rows)

    plane = (nt // 128, 128)
    bidx_p = bidx.reshape(plane)
    flag_p = jnp.broadcast_to(
        approx_flag[t].astype(jnp.float32)[:, None], (N, L)).reshape(plane)
    std_p = jnp.broadcast_to(stddevs[t][:, None], (N, L)).reshape(plane)
    ab_p = jnp.broadcast_to(alpha_bars[t][:, None], (N, L)).reshape(plane)
    v0x = v_0[..., 0].reshape(plane)
    v0y = v_0[..., 1].reshape(plane)
    v0z = v_0[..., 2].reshape(plane)

    outs = pl.pallas_call(
        _epi_kernel,
        out_shape=[jax.ShapeDtypeStruct(plane, jnp.float32)] * 6,
    )(bidx_p, flag_p, std_p, ab_p, v0x, v0y, v0z)
    vx, vy, vz, ex, ey, ez = outs
    v_noisy = jnp.stack(
        [vx.reshape(nt), vy.reshape(nt), vz.reshape(nt)], axis=-1
    ).reshape(N, L, 3)
    e_scaled = jnp.stack(
        [ex.reshape(nt), ey.reshape(nt), ez.reshape(nt)], axis=-1
    ).reshape(N, L, 3)
    return v_noisy, e_scaled


# speedup vs baseline: 1.8313x; 1.0799x over previous
import math

import numpy as np
import jax
import jax.numpy as jnp
from jax.experimental import pallas as pl
from jax.experimental.pallas import tpu as pltpu

_NUM_BINS = 8192
_B = _NUM_BINS - 1
_PI = math.pi
_TINY = float(np.finfo(np.float32).tiny)
_FMAX = 3.0e38
_LO = float(np.nextafter(np.float32(-1.0), np.float32(0.0)))
_SPAN = float(np.float32(np.float32(1.0) - np.float32(_LO)))
_SQRT2 = float(np.float32(np.sqrt(2.0)))
_STEP = float(np.float32(_PI / _B))

_TOKT = 64
_CHUNK = 4096
_NCH = _NUM_BINS // _CHUNK


def _tf_host(k0, k1, x0, x1):
    msk = 0xFFFFFFFF
    ks = (k0, k1, (k0 ^ k1 ^ 0x1BD11BDA) & msk)
    rot = ((13, 15, 26, 6), (17, 29, 16, 24))
    inj = ((ks[1], ks[2], 1), (ks[2], ks[0], 2), (ks[0], ks[1], 3),
           (ks[1], ks[2], 4), (ks[2], ks[0], 5))
    x0 = (x0 + ks[0]) & msk
    x1 = (x1 + ks[1]) & msk
    for i in range(5):
        for r in rot[i % 2]:
            x0 = (x0 + x1) & msk
            x1 = ((x1 << r) | (x1 >> (32 - r))) & msk
            x1 ^= x0
        a, b, c = inj[i]
        x0 = (x0 + a) & msk
        x1 = (x1 + b + c) & msk
    return x0, x1


def _split_host(key, num):
    return [_tf_host(key[0], key[1], 0, i) for i in range(num)]


_KEY_ROOT = (0, 1)
_KU, _KT = _split_host(_KEY_ROOT, 2)
_K1, _K2, _K3 = _split_host(_KT, 3)


def _i32c(v):
    return jnp.int32(int(np.uint32(v & 0xFFFFFFFF).astype(np.int32)))


def _threefry_bits(key, x1):
    k0, k1 = key
    ks2 = (k0 ^ k1 ^ 0x1BD11BDA) & 0xFFFFFFFF
    rot = ((13, 15, 26, 6), (17, 29, 16, 24))
    inj = ((k1, ks2, 1), (ks2, k0, 2), (k0, k1, 3), (k1, ks2, 4), (ks2, k0, 5))
    x0 = jnp.full(x1.shape, _i32c(k0), jnp.int32)
    x1 = x1 + _i32c(k1)
    for i in range(5):
        for r in rot[i % 2]:
            x0 = x0 + x1
            x1 = jax.lax.shift_left(x1, jnp.int32(r)) | \
                jax.lax.shift_right_logical(x1, jnp.int32(32 - r))
            x1 = x1 ^ x0
        a, b, c = inj[i]
        x0 = x0 + _i32c(a)
        x1 = x1 + _i32c(b + c)
    return x0 ^ x1


def _bits_to_unit(bits):
    fb = jax.lax.shift_right_logical(bits, jnp.int32(9)) | jnp.int32(0x3F800000)
    return jax.lax.bitcast_convert_type(fb, jnp.float32) - jnp.float32(1.0)


def _horner(w, coeffs):
    p = jnp.float32(coeffs[0])
    for c in coeffs[1:]:
        p = p * w + jnp.float32(c)
    return p


_ERFINV_SMALL = (2.81022636e-08, 3.43273939e-07, -3.5233877e-06,
                 -4.39150654e-06, 0.00021858087, -0.00125372503,
                 -0.00417768164, 0.246640727, 1.50140941)
_ERFINV_BIG = (-0.000200214257, 0.000100950558, 0.00134934322,
               -0.00367342844, 0.00573950773, -0.0076224613,
               0.00943887047, 1.00167406, 2.83297682)


def _erfinv(x):
    w = -jnp.log((jnp.float32(1.0) - x) * (jnp.float32(1.0) + x))
    small = w < jnp.float32(5.0)
    ws = w - jnp.float32(2.5)
    wb = jnp.sqrt(jnp.where(small, jnp.float32(5.0), w)) - jnp.float32(3.0)
    p = jnp.where(small, _horner(ws, _ERFINV_SMALL), _horner(wb, _ERFINV_BIG))
    return p * x


_ACOS_C = (-0.0012624911, 0.0066700901, -0.0170881256, 0.0308918810,
           -0.0501743046, 0.0889789874, -0.2145988016, 1.5707963050)


def _acos(x):
    ax = jnp.abs(x)
    p = _horner(ax, _ACOS_C)
    r = jnp.sqrt(jnp.maximum(jnp.float32(1.0) - ax, jnp.float32(0.0))) * p
    return jnp.where(x >= 0, r, jnp.float32(_PI) - r)


def _std_normal(key, m):
    f = _bits_to_unit(_threefry_bits(key, m))
    u = jnp.maximum(f * jnp.float32(_SPAN) + jnp.float32(_LO), jnp.float32(_LO))
    return jnp.float32(_SQRT2) * _erfinv(u)


def _bf(x):
    return x.astype(jnp.bfloat16).astype(jnp.float32)


def _rodrigues(x, y, z):
    n = jnp.sqrt(x * x + y * y + z * z)
    s = jnp.sin(n)
    ct = jnp.cos(n)
    b = (s + jnp.float32(1e-8)) / (n + jnp.float32(1e-8))
    c = (jnp.float32(1.0) - ct + jnp.float32(1e-8)) / (n * n + jnp.float32(2e-8))
    xb = _bf(x)
    yb = _bf(y)
    zb = _bf(z)
    s2xy = xb * yb
    s2xz = xb * zb
    s2yz = yb * zb
    r00 = 1.0 + c * (-(zb * zb) - yb * yb)
    r01 = b * z + c * s2xy
    r02 = -(b * y) + c * s2xz
    r10 = -(b * z) + c * s2xy
    r11 = 1.0 + c * (-(zb * zb) - xb * xb)
    r12 = b * x + c * s2yz
    r20 = b * y + c * s2xz
    r21 = -(b * x) + c * s2yz
    r22 = 1.0 + c * (-(yb * yb) - xb * xb)
    return (r00, r01, r02, r10, r11, r12, r20, r21, r22)


_SLICE = 8


def _cat_kernel(t_ref, f_ref, y_ref, bidx_ref, bw_ref, bb_ref):
    p = pl.program_id(0)
    k = pl.program_id(1)
    jrow = jax.lax.broadcasted_iota(jnp.int32, (1, _CHUNK), 1)
    nrecip = jnp.float32(-1.0) / (y_ref[0] + jnp.float32(1e-20))
    nrecip = jnp.where(k * _CHUNK + jrow >= _B, jnp.float32(-_FMAX), nrecip)

    @pl.when(k == 0)
    def _init():
        bw_ref[...] = jnp.full((_TOKT, _CHUNK), _FMAX, jnp.float32)
        bb_ref[...] = jnp.zeros((_TOKT, _CHUNK), jnp.int32)

    @pl.when(f_ref[p * _TOKT // 256] == 0)
    def _race():
        for ss in range(_TOKT // _SLICE):
            l = jax.lax.broadcasted_iota(
                jnp.int32, (_SLICE, _CHUNK), 0) + ss * _SLICE
            j = jax.lax.broadcasted_iota(jnp.int32, (_SLICE, _CHUNK), 1)
            gbin = k * _CHUNK + j
            m = (p * _TOKT + l) * _B + gbin
            bits = _threefry_bits(_K1, m)
            u = jnp.maximum(_bits_to_unit(bits), jnp.float32(_TINY))
            w = jnp.log(u) * nrecip
            sl = slice(ss * _SLICE, (ss + 1) * _SLICE)
            bwv = bw_ref[sl, :]
            bbv = bb_ref[sl, :]
            upd = w < bwv
            bb_ref[sl, :] = jnp.where(upd, gbin, bbv)
            bw_ref[sl, :] = jnp.where(upd, w, bwv)

    @pl.when(k == _NCH - 1)
    def _finish():
        wv = bw_ref[...]
        mv = jnp.min(wv, axis=1, keepdims=True)
        cand = jnp.where(wv == mv, bb_ref[...], jnp.int32(1 << 30))
        bidx_ref[0] = jnp.min(cand, axis=1, keepdims=True)


def _epi_kernel(bidx_ref, flag_ref, std_ref, ab_ref, v0x_ref, v0y_ref, v0z_ref,
                vx_ref, vy_ref, vz_ref, ex_ref, ey_ref, ez_ref):
    shape = (64, 128)
    i = jax.lax.broadcasted_iota(jnp.int32, shape, 0)
    j = jax.lax.broadcasted_iota(jnp.int32, shape, 1)
    r = i * 128 + j

    u2 = _bits_to_unit(_threefry_bits(_K2, r))
    bf = bidx_ref[...].astype(jnp.float32)
    smp_h = bf * jnp.float32(_STEP) + u2 * jnp.float32(_STEP)

    n3 = _std_normal(_K3, r)
    std = std_ref[...]
    smp_g = jax.lax.rem(jnp.abs(std * jnp.float32(2.0) + n3 * std),
                        jnp.float32(_PI))
    theta = jnp.where(flag_ref[...] > jnp.float32(0.5), smp_g, smp_h)

    m0 = r * 3
    g0 = _std_normal(_KU, m0)
    g1 = _std_normal(_KU, m0 + 1)
    g2 = _std_normal(_KU, m0 + 2)
    nrm = jnp.sqrt(g0 * g0 + g1 * g1 + g2 * g2) + jnp.float32(1e-12)
    ex = (g0 / nrm) * theta
    ey = (g1 / nrm) * theta
    ez = (g2 / nrm) * theta
    ex_ref[...] = ex
    ey_ref[...] = ey
    ez_ref[...] = ez

    E = [_bf(v) for v in _rodrigues(ex, ey, ez)]
    c0 = jnp.sqrt(ab_ref[...])
    A = [_bf(v) for v in _rodrigues(
        c0 * v0x_ref[...], c0 * v0y_ref[...], c0 * v0z_ref[...])]
    e00, e01, e02, e10, e11, e12, e20, e21, e22 = E
    a00, a01, a02, a10, a11, a12, a20, a21, a22 = A
    r00 = e00 * a00 + e01 * a10 + e02 * a20
    r01 = e00 * a01 + e01 * a11 + e02 * a21
    r02 = e00 * a02 + e01 * a12 + e02 * a22
    r10 = e10 * a00 + e11 * a10 + e12 * a20
    r11 = e10 * a01 + e11 * a11 + e12 * a21
    r12 = e10 * a02 + e11 * a12 + e12 * a22
    r20 = e20 * a00 + e21 * a10 + e22 * a20
    r21 = e20 * a01 + e21 * a11 + e22 * a21
    r22 = e20 * a02 + e21 * a12 + e22 * a22

    tr = r00 + r11 + r22
    ct = jnp.clip((tr - jnp.float32(1.0)) / jnp.float32(2.0),
                  jnp.float32(-0.999), jnp.float32(1.0))
    st = jnp.sqrt(jnp.float32(1.0) - ct * ct)
    th = _acos(ct)
    coef = (th + jnp.float32(1e-8)) / (jnp.float32(2.0) * st + jnp.float32(2e-8))
    vx_ref[...] = coef * (r12 - r21)
    vy_ref[...] = coef * (r20 - r02)
    vz_ref[...] = coef * (r01 - r10)


def kernel(v_0, t, X, Y, stddevs, approx_flag, alpha_bars):
    N, L = v_0.shape[:2]
    nt = N * L
    t = t.astype(jnp.int32)
    tiles = nt // _TOKT

    y_rows = Y.reshape(Y.shape[0] * _NCH, 1, _CHUNK)
    skip_seq = approx_flag[t].astype(jnp.int32)

    grid_spec = pltpu.PrefetchScalarGridSpec(
        num_scalar_prefetch=2,
        grid=(tiles, _NCH),
        in_specs=[
            pl.BlockSpec(
                (1, 1, _CHUNK),
                lambda p, k, t_ref, f_ref:
                    (t_ref[(p * _TOKT) // L] * _NCH + k, 0, 0)),
        ],
        out_specs=pl.BlockSpec(
            (1, _TOKT, 1), lambda p, k, t_ref, f_ref: (p, 0, 0)),
        scratch_shapes=[
            pltpu.VMEM((_TOKT, _CHUNK), jnp.float32),
            pltpu.VMEM((_TOKT, _CHUNK), jnp.int32),
        ],
    )
    bidx = pl.pallas_call(
        _cat_kernel,
        grid_spec=grid_spec,
        out_shape=jax.ShapeDtypeStruct((tiles, _TOKT, 1), jnp.int32),
        compiler_params=pltpu.CompilerParams(
            dimension_semantics=("arbitrary", "arbitrary")),
    )(t, skip_seq, y_rows)

    plane = (nt // 128, 128)
    bidx_p = bidx.reshape(plane)
    flag_p = jnp.broadcast_to(
        approx_flag[t].astype(jnp.float32)[:, None], (N, L)).reshape(plane)
    std_p = jnp.broadcast_to(stddevs[t][:, None], (N, L)).reshape(plane)
    ab_p = jnp.broadcast_to(alpha_bars[t][:, None], (N, L)).reshape(plane)
    v0x = v_0[..., 0].reshape(plane)
    v0y = v_0[..., 1].reshape(plane)
    v0z = v_0[..., 2].reshape(plane)

    outs = pl.pallas_call(
        _epi_kernel,
        out_shape=[jax.ShapeDtypeStruct(plane, jnp.float32)] * 6,
    )(bidx_p, flag_p, std_p, ab_p, v0x, v0y, v0z)
    vx, vy, vz, ex, ey, ez = outs
    v_noisy = jnp.stack(
        [vx.reshape(nt), vy.reshape(nt), vz.reshape(nt)], axis=-1
    ).reshape(N, L, 3)
    e_scaled = jnp.stack(
        [ex.reshape(nt), ey.reshape(nt), ez.reshape(nt)], axis=-1
    ).reshape(N, L, 3)
    return v_noisy, e_scaled
